# baseline (speedup 1.0000x reference)
import functools

import jax
import jax.numpy as jnp
from jax import lax
from jax.experimental import pallas as pl
from jax.experimental.pallas import tpu as pltpu
from jax.experimental.pallas import tpu_sc as plsc

NC = 2
NS = 16
L = 16

N_PAD = 100352
ROW_W = 64
CHUNK_ROWS = 64


def _occupancy_call(idx2d):
    rows = idx2d.shape[0]
    nw = NC * NS
    total_chunks = (rows + CHUNK_ROWS - 1) // CHUNK_ROWS
    nloop = (total_chunks + nw - 1) // nw
    npair = (nloop + 1) // 2
    last_r0 = (max(rows - CHUNK_ROWS, 0) // 8) * 8
    slice_n = N_PAD // NS
    mesh = plsc.VectorSubcoreMesh(core_axis_name="c", subcore_axis_name="s")

    @functools.partial(
        pl.kernel,
        mesh=mesh,
        out_type=jax.ShapeDtypeStruct((NC, N_PAD), jnp.float32),
        scratch_types=[
            pltpu.VMEM((2, CHUNK_ROWS, ROW_W), jnp.int32),
            pltpu.VMEM((ROW_W,), jnp.float32),
            pltpu.VMEM((slice_n,), jnp.float32),
            pltpu.VMEM_SHARED((N_PAD,), jnp.float32),
            pltpu.SemaphoreType.DMA,
            pltpu.SemaphoreType.DMA,
            pltpu.SemaphoreType.DMA,
        ],
    )
    def occ_kernel(idx_hbm, out_hbm, idx_v, ones_v, zero_v, acc_sh,
                   lsem0, lsem1, ssem):
        c = lax.axis_index("c")
        s = lax.axis_index("s")
        wid = s * NC + c
        lsems = (lsem0, lsem1)
        for k in range(ROW_W // L):
            ones_v[pl.ds(k * L, L)] = jnp.ones((L,), jnp.float32)

        def zbody(i, carry):
            zero_v[pl.ds(i * L, L)] = jnp.zeros((L,), jnp.float32)
            return carry

        lax.fori_loop(0, slice_n // L, zbody, 0)
        pltpu.sync_copy(zero_v, acc_sh.at[pl.ds(s * slice_n, slice_n)])
        plsc.subcore_barrier()

        def start_load(i, buf):
            cc = wid + i * nw

            @pl.when(cc < total_chunks)
            def _():
                r0 = pl.multiple_of(
                    jnp.minimum(cc * CHUNK_ROWS, last_r0), 8)
                pltpu.async_copy(idx_hbm.at[pl.ds(r0, CHUNK_ROWS)],
                                 idx_v.at[buf], lsems[buf])

        start_load(0, 0)
        start_load(1, 1)

        def pair_body(i2, carry):
            for k in (0, 1):
                i = 2 * i2 + k
                cc = wid + i * nw

                @pl.when(cc < total_chunks)
                def _():
                    pltpu.make_async_copy(
                        idx_hbm.at[pl.ds(0, CHUNK_ROWS)],
                        idx_v.at[k], lsems[k]).wait()
                    hs = [pltpu.async_copy(
                              ones_v, acc_sh.at[idx_v.at[k, j]], ssem,
                              add=True)
                          for j in range(CHUNK_ROWS)]
                    for h in hs:
                        h.wait()
                    start_load(i + 2, k)

            return carry

        lax.fori_loop(0, npair, pair_body, 0)
        plsc.subcore_barrier()
        pltpu.sync_copy(acc_sh.at[pl.ds(s * slice_n, slice_n)],
                        out_hbm.at[c, pl.ds(s * slice_n, slice_n)])

    return occ_kernel(idx2d)


def _gdb_mask_body(xs_ref, xv_ref, occa_ref, occb_ref,
                   a1_ref, a2_ref, s3_ref, r8_ref,
                   ws1_ref, w2n_ref, w2z_ref, wg_ref, bg_ref,
                   outs_ref, outv_ref):
    f32 = jnp.float32
    dot = functools.partial(jnp.dot, preferred_element_type=f32)
    bn = xs_ref.shape[0]
    xs = xs_ref[...]
    xv = xv_ref[...]
    vi = dot(xv, a1_ref[...])
    vn = jnp.sqrt(dot(vi * vi, s3_ref[...]))
    z = dot(xs, ws1_ref[...])
    os_ = dot(vn, w2n_ref[...]) + dot(z, w2z_ref[...])
    gl = dot(os_, wg_ref[...]) + bg_ref[...]
    gate = 1.0 / (1.0 + jnp.exp(-gl))
    ov = dot(vi, a2_ref[...]) * dot(gate, r8_ref[...])
    mask = (occa_ref[...] + occb_ref[...]).reshape(bn, 1) > 0.0
    outs_ref[...] = jnp.where(mask, os_, 0.0)
    outv_ref[...] = jnp.where(mask, ov, 0.0)


def kernel(x_sca, x_vec, query_sca, query_vec, edge_index_i,
           Wk_vec1, Wk_vec2, Wk_s1, Wk_s2, Wk_gate, bk_gate,
           Wv_vec1, Wv_vec2, Wv_s1, Wv_s2, Wv_gate, bv_gate):
    n = x_sca.shape[0]
    e = edge_index_i.shape[0]

    if e % ROW_W:
        pad = ROW_W - e % ROW_W
        edge_index_i = jnp.concatenate(
            [edge_index_i, jnp.full((pad,), n, jnp.int32)])
    idx2d = edge_index_i.reshape(-1, ROW_W)
    occ2 = _occupancy_call(idx2d)
    occ_a = occ2[0, :n]
    occ_b = occ2[1, :n]

    eye3 = jnp.eye(3, dtype=jnp.float32)
    a1 = jnp.kron(Wv_vec1, eye3).T
    a2 = jnp.kron(Wv_vec2, eye3).T
    s3 = jnp.kron(jnp.eye(8, dtype=jnp.float32), jnp.ones((3, 1), jnp.float32))
    r8 = jnp.kron(jnp.eye(8, dtype=jnp.float32), jnp.ones((1, 3), jnp.float32))
    ws1 = Wv_s1.T
    w2n = Wv_s2.T[:8]
    w2z = Wv_s2.T[8:]
    wg = Wv_gate.T
    bg = bv_gate.reshape(1, 8)

    bn = 2048
    grid = ((n + bn - 1) // bn,)
    full = lambda i: (0, 0)
    row = lambda i: (i, 0)
    out_s, out_v = pl.pallas_call(
        _gdb_mask_body,
        grid=grid,
        in_specs=[
            pl.BlockSpec((bn, 32), row),
            pl.BlockSpec((bn, 24), row),
            pl.BlockSpec((bn,), lambda i: (i,)),
            pl.BlockSpec((bn,), lambda i: (i,)),
            pl.BlockSpec((24, 24), full),
            pl.BlockSpec((24, 24), full),
            pl.BlockSpec((24, 8), full),
            pl.BlockSpec((8, 24), full),
            pl.BlockSpec((32, 8), full),
            pl.BlockSpec((8, 16), full),
            pl.BlockSpec((8, 16), full),
            pl.BlockSpec((16, 8), full),
            pl.BlockSpec((1, 8), full),
        ],
        out_specs=[
            pl.BlockSpec((bn, 16), row),
            pl.BlockSpec((bn, 24), row),
        ],
        out_shape=[
            jax.ShapeDtypeStruct((n, 16), jnp.float32),
            jax.ShapeDtypeStruct((n, 24), jnp.float32),
        ],
    )(x_sca, x_vec.reshape(n, 24), occ_a, occ_b,
      a1, a2, s3, r8, ws1, w2n, w2z, wg, bg)

    return out_s, out_v.reshape(n, 8, 3)


# speedup vs baseline: 1.0306x; 1.0306x over previous
import functools

import jax
import jax.numpy as jnp
from jax import lax
from jax.experimental import pallas as pl
from jax.experimental.pallas import tpu as pltpu
from jax.experimental.pallas import tpu_sc as plsc

NC = 2
NS = 16
L = 16

N_PAD = 100352
ROW_W = 128
CHUNK_ROWS = 32


def _occupancy_call(idx2d):
    rows = idx2d.shape[0]
    nw = NC * NS
    total_chunks = (rows + CHUNK_ROWS - 1) // CHUNK_ROWS
    nloop = (total_chunks + nw - 1) // nw
    npair = (nloop + 1) // 2
    last_r0 = (max(rows - CHUNK_ROWS, 0) // 8) * 8
    tail_rows = rows - min(last_r0 + CHUNK_ROWS, rows)
    if tail_rows:
        tail_rows = rows - (rows // 8) * 8
        tail_r0 = (rows // 8) * 8
    slice_n = N_PAD // NS
    mesh = plsc.VectorSubcoreMesh(core_axis_name="c", subcore_axis_name="s")

    @functools.partial(
        pl.kernel,
        mesh=mesh,
        out_type=jax.ShapeDtypeStruct((NC, N_PAD), jnp.float32),
        scratch_types=[
            pltpu.VMEM((2, CHUNK_ROWS, ROW_W), jnp.int32),
            pltpu.VMEM((ROW_W,), jnp.float32),
            pltpu.VMEM((slice_n,), jnp.float32),
            pltpu.VMEM_SHARED((N_PAD,), jnp.float32),
            pltpu.SemaphoreType.DMA,
            pltpu.SemaphoreType.DMA,
            pltpu.SemaphoreType.DMA,
        ],
    )
    def occ_kernel(idx_hbm, out_hbm, idx_v, ones_v, zero_v, acc_sh,
                   lsem0, lsem1, ssem):
        c = lax.axis_index("c")
        s = lax.axis_index("s")
        wid = s * NC + c
        lsems = (lsem0, lsem1)
        for k in range(ROW_W // L):
            ones_v[pl.ds(k * L, L)] = jnp.ones((L,), jnp.float32)

        def zbody(i, carry):
            zero_v[pl.ds(i * L, L)] = jnp.zeros((L,), jnp.float32)
            return carry

        lax.fori_loop(0, slice_n // L, zbody, 0)
        pltpu.sync_copy(zero_v, acc_sh.at[pl.ds(s * slice_n, slice_n)])
        plsc.subcore_barrier()

        if tail_rows:
            @pl.when(wid == 0)
            def _():
                pltpu.sync_copy(idx_hbm.at[pl.ds(tail_r0, tail_rows)],
                                idx_v.at[0, pl.ds(0, tail_rows)])
                for j in range(tail_rows):
                    pltpu.sync_copy(ones_v, acc_sh.at[idx_v.at[0, j]],
                                    add=True)

        def start_load(i, buf):
            cc = wid + i * nw

            @pl.when(cc < total_chunks)
            def _():
                r0 = pl.multiple_of(
                    jnp.minimum(cc * CHUNK_ROWS, last_r0), 8)
                pltpu.async_copy(idx_hbm.at[pl.ds(r0, CHUNK_ROWS)],
                                 idx_v.at[buf], lsems[buf])

        start_load(0, 0)
        start_load(1, 1)

        def pair_body(i2, carry):
            for k in (0, 1):
                i = 2 * i2 + k
                cc = wid + i * nw

                @pl.when(cc < total_chunks)
                def _():
                    pltpu.make_async_copy(
                        idx_hbm.at[pl.ds(0, CHUNK_ROWS)],
                        idx_v.at[k], lsems[k]).wait()
                    hs = [pltpu.async_copy(
                              ones_v, acc_sh.at[idx_v.at[k, j]], ssem,
                              add=True)
                          for j in range(CHUNK_ROWS)]
                    for h in hs:
                        h.wait()
                    start_load(i + 2, k)

            return carry

        lax.fori_loop(0, npair, pair_body, 0)
        plsc.subcore_barrier()
        pltpu.sync_copy(acc_sh.at[pl.ds(s * slice_n, slice_n)],
                        out_hbm.at[c, pl.ds(s * slice_n, slice_n)])

    return occ_kernel(idx2d)


def _gdb_mask_body(xs_ref, xv_ref, occa_ref, occb_ref,
                   a1_ref, a2_ref, s3_ref, r8_ref,
                   ws1_ref, w2n_ref, w2z_ref, wg_ref, bg_ref,
                   outs_ref, outv_ref):
    f32 = jnp.float32
    dot = functools.partial(jnp.dot, preferred_element_type=f32)
    bn = xs_ref.shape[0]
    xs = xs_ref[...]
    xv = xv_ref[...]
    vi = dot(xv, a1_ref[...])
    vn = jnp.sqrt(dot(vi * vi, s3_ref[...]))
    z = dot(xs, ws1_ref[...])
    os_ = dot(vn, w2n_ref[...]) + dot(z, w2z_ref[...])
    gl = dot(os_, wg_ref[...]) + bg_ref[...]
    gate = 1.0 / (1.0 + jnp.exp(-gl))
    ov = dot(vi, a2_ref[...]) * dot(gate, r8_ref[...])
    mask = (occa_ref[...] + occb_ref[...]).reshape(bn, 1) > 0.0
    outs_ref[...] = jnp.where(mask, os_, 0.0)
    outv_ref[...] = jnp.where(mask, ov, 0.0)


def kernel(x_sca, x_vec, query_sca, query_vec, edge_index_i,
           Wk_vec1, Wk_vec2, Wk_s1, Wk_s2, Wk_gate, bk_gate,
           Wv_vec1, Wv_vec2, Wv_s1, Wv_s2, Wv_gate, bv_gate):
    n = x_sca.shape[0]
    e = edge_index_i.shape[0]

    if e % ROW_W:
        pad = ROW_W - e % ROW_W
        edge_index_i = jnp.concatenate(
            [edge_index_i, jnp.full((pad,), n, jnp.int32)])
    idx2d = edge_index_i.reshape(-1, ROW_W)
    occ2 = _occupancy_call(idx2d)
    occ_a = occ2[0, :n]
    occ_b = occ2[1, :n]

    eye3 = jnp.eye(3, dtype=jnp.float32)
    a1 = jnp.kron(Wv_vec1, eye3).T
    a2 = jnp.kron(Wv_vec2, eye3).T
    s3 = jnp.kron(jnp.eye(8, dtype=jnp.float32), jnp.ones((3, 1), jnp.float32))
    r8 = jnp.kron(jnp.eye(8, dtype=jnp.float32), jnp.ones((1, 3), jnp.float32))
    ws1 = Wv_s1.T
    w2n = Wv_s2.T[:8]
    w2z = Wv_s2.T[8:]
    wg = Wv_gate.T
    bg = bv_gate.reshape(1, 8)

    bn = 2048
    grid = ((n + bn - 1) // bn,)
    full = lambda i: (0, 0)
    row = lambda i: (i, 0)
    out_s, out_v = pl.pallas_call(
        _gdb_mask_body,
        grid=grid,
        in_specs=[
            pl.BlockSpec((bn, 32), row),
            pl.BlockSpec((bn, 24), row),
            pl.BlockSpec((bn,), lambda i: (i,)),
            pl.BlockSpec((bn,), lambda i: (i,)),
            pl.BlockSpec((24, 24), full),
            pl.BlockSpec((24, 24), full),
            pl.BlockSpec((24, 8), full),
            pl.BlockSpec((8, 24), full),
            pl.BlockSpec((32, 8), full),
            pl.BlockSpec((8, 16), full),
            pl.BlockSpec((8, 16), full),
            pl.BlockSpec((16, 8), full),
            pl.BlockSpec((1, 8), full),
        ],
        out_specs=[
            pl.BlockSpec((bn, 16), row),
            pl.BlockSpec((bn, 24), row),
        ],
        out_shape=[
            jax.ShapeDtypeStruct((n, 16), jnp.float32),
            jax.ShapeDtypeStruct((n, 24), jnp.float32),
        ],
    )(x_sca, x_vec.reshape(n, 24), occ_a, occ_b,
      a1, a2, s3, r8, ws1, w2n, w2z, wg, bg)

    return out_s, out_v.reshape(n, 8, 3)
